# Initial kernel scaffold; baseline (speedup 1.0000x reference)
import jax
import jax.numpy as jnp
from jax.experimental import pallas as pl


def kernel(attn):
    raise NotImplementedError("write your pallas kernel here")


# speedup vs baseline: 17.2853x; 17.2853x over previous
import jax
import jax.numpy as jnp
from jax.experimental import pallas as pl
from jax.experimental.pallas import tpu as pltpu

_TOPK = 64
_ROW_BLOCK = 32


def _topk_mask_body(x_ref, o_ref):
    x = x_ref[...]
    u = jax.lax.bitcast_convert_type(x, jnp.uint32)
    key = jnp.where(
        (u >> 31) == 1, ~u, u | jnp.uint32(0x80000000)
    )

    prefix0 = jnp.zeros((x.shape[0], 1), dtype=jnp.uint32)

    def step(b, prefix):
        cand = prefix | (jnp.uint32(1) << (31 - b).astype(jnp.uint32))
        cnt = jnp.sum((key >= cand).astype(jnp.int32), axis=1, keepdims=True)
        return jnp.where(cnt >= _TOPK, cand, prefix)

    thresh = jax.lax.fori_loop(0, 32, step, prefix0, unroll=True)
    o_ref[...] = jnp.where(key >= thresh, x, jnp.float32(0.0))


def kernel(attn):
    mb, num_q, num_k = attn.shape
    rows = mb * num_q
    flat = attn.reshape(rows, num_k)
    out = pl.pallas_call(
        _topk_mask_body,
        grid=(rows // _ROW_BLOCK,),
        in_specs=[pl.BlockSpec((_ROW_BLOCK, num_k), lambda r: (r, 0))],
        out_specs=pl.BlockSpec((_ROW_BLOCK, num_k), lambda r: (r, 0)),
        out_shape=jax.ShapeDtypeStruct((rows, num_k), jnp.float32),
    )(flat)
    return out.reshape(mb, num_q, num_k)
